# Initial kernel scaffold; baseline (speedup 1.0000x reference)
import jax
import jax.numpy as jnp
from jax.experimental import pallas as pl


def kernel(atom_feat, degree, W_atom, W_degree, W_vnode):
    raise NotImplementedError("write your pallas kernel here")


# speedup vs baseline: 8.6269x; 8.6269x over previous
import functools

import jax
import jax.numpy as jnp
from jax import lax
from jax.experimental import pallas as pl
from jax.experimental.pallas import tpu as pltpu
from jax.experimental.pallas import tpu_sc as plsc

G = 1024
N = 64
F = 9
H = 64
NP1 = N + 1
L = 16

NC = 2
NS = 16
NW = NC * NS
GPW = G // NW

ACH = 8
ACW = (N * F) // ACH


@functools.partial(
    pl.kernel,
    mesh=plsc.VectorSubcoreMesh(core_axis_name="c", subcore_axis_name="s"),
    out_type=jax.ShapeDtypeStruct((G, NP1, H), jnp.float32),
    scratch_types=[
        pltpu.VMEM((ACH, ACW), jnp.int32),
        pltpu.VMEM((N,), jnp.int32),
        pltpu.VMEM((N * F, H), jnp.float32),
        pltpu.VMEM((N, H), jnp.float32),
        pltpu.VMEM((NP1, H), jnp.float32),
        pltpu.SemaphoreType.DMA,
    ],
    compiler_params=pltpu.CompilerParams(use_tc_tiling_on_sc=False),
)
def _atom_feature_sc(af_hbm, dg_hbm, wa_hbm, wd_hbm, wv_hbm, out_hbm,
                     aidx_v, didx_v, arows_v, drows_v, obuf_v, sem):
    wid = lax.axis_index("s") * NC + lax.axis_index("c")

    pltpu.sync_copy(wv_hbm, obuf_v.at[pl.ds(0, 1)])

    def per_graph(gl, carry):
        g = wid * GPW + gl
        pltpu.sync_copy(af_hbm.at[g], aidx_v)
        pltpu.sync_copy(dg_hbm.at[g], didx_v)
        cps = [
            pltpu.async_copy(wa_hbm.at[aidx_v.at[i]],
                             arows_v.at[pl.ds(i * ACW, ACW)], sem)
            for i in range(ACH)
        ]
        cps.append(pltpu.async_copy(wd_hbm.at[didx_v], drows_v, sem))
        for cp in cps:
            cp.wait()

        def per_node(n, carry2):
            base = n * F
            for j in range(H // L):
                sl = pl.ds(j * L, L)
                acc = drows_v[n, sl]
                for f in range(F):
                    acc = acc + arows_v[base + f, sl]
                obuf_v[n + 1, sl] = acc
            return carry2

        lax.fori_loop(0, N, per_node, 0)
        pltpu.sync_copy(obuf_v, out_hbm.at[g])
        return carry

    lax.fori_loop(0, GPW, per_graph, 0)


def kernel(atom_feat, degree, W_atom, W_degree, W_vnode):
    af = atom_feat.reshape(G, ACH, ACW)
    return _atom_feature_sc(af, degree, W_atom, W_degree, W_vnode)


# speedup vs baseline: 14.0936x; 1.6337x over previous
import functools

import jax
import jax.numpy as jnp
from jax import lax
from jax.experimental import pallas as pl
from jax.experimental.pallas import tpu as pltpu
from jax.experimental.pallas import tpu_sc as plsc

G = 1024
N = 64
F = 9
H = 64
NP1 = N + 1

NC = 2
NS = 16
NW = NC * NS
GPW = G // NW
BG = 16
NB = GPW // BG


@functools.partial(
    pl.kernel,
    mesh=plsc.VectorSubcoreMesh(core_axis_name="c", subcore_axis_name="s"),
    out_type=jax.ShapeDtypeStruct((G, NP1, H), jnp.float32),
    scratch_types=[
        pltpu.VMEM((BG, F, N), jnp.int32),
        pltpu.VMEM((BG, N), jnp.int32),
        pltpu.VMEM((BG, NP1, H), jnp.float32),
        pltpu.SemaphoreType.DMA,
        pltpu.SemaphoreType.DMA,
    ],
    compiler_params=pltpu.CompilerParams(use_tc_tiling_on_sc=False),
)
def _atom_feature_sc(af_hbm, dg_hbm, wa_hbm, wd_hbm, wv_hbm, out_hbm,
                     aidx_v, didx_v, obuf_v, sem, sem2):
    wid = lax.axis_index("s") * NC + lax.axis_index("c")

    for k in range(BG):
        pltpu.async_copy(wv_hbm, obuf_v.at[k, pl.ds(0, 1)], sem2)
    for k in range(BG):
        pltpu.make_async_copy(wv_hbm, obuf_v.at[k, pl.ds(0, 1)], sem2).wait()

    def per_batch(b, carry):
        g0 = wid * GPW + b * BG
        pltpu.async_copy(af_hbm.at[pl.ds(g0, BG)], aidx_v, sem2)
        pltpu.async_copy(dg_hbm.at[pl.ds(g0, BG)], didx_v, sem2)
        pltpu.make_async_copy(af_hbm.at[pl.ds(g0, BG)], aidx_v, sem2).wait()
        pltpu.make_async_copy(dg_hbm.at[pl.ds(g0, BG)], didx_v, sem2).wait()

        def issue_deg(k, c):
            pltpu.async_copy(wd_hbm.at[didx_v.at[k]],
                             obuf_v.at[k, pl.ds(1, N)], sem)
            return c
        lax.fori_loop(0, BG, issue_deg, 0)

        def drain_deg(k, c):
            pltpu.make_async_copy(wd_hbm.at[didx_v.at[k]],
                                  obuf_v.at[k, pl.ds(1, N)], sem).wait()
            return c
        lax.fori_loop(0, BG, drain_deg, 0)

        def issue_atom(k, c):
            for f in range(F):
                pltpu.async_copy(wa_hbm.at[aidx_v.at[k, f]],
                                 obuf_v.at[k, pl.ds(1, N)], sem, add=True)
            return c
        lax.fori_loop(0, BG, issue_atom, 0)

        def drain_atom(k, c):
            for f in range(F):
                pltpu.make_async_copy(wa_hbm.at[aidx_v.at[k, f]],
                                      obuf_v.at[k, pl.ds(1, N)], sem).wait()
            return c
        lax.fori_loop(0, BG, drain_atom, 0)

        pltpu.sync_copy(obuf_v, out_hbm.at[pl.ds(g0, BG)])
        return carry

    lax.fori_loop(0, NB, per_batch, 0)


def kernel(atom_feat, degree, W_atom, W_degree, W_vnode):
    af_t = atom_feat.transpose(0, 2, 1)
    return _atom_feature_sc(af_t, degree, W_atom, W_degree, W_vnode)
